# baseline (speedup 1.0000x reference)
import functools

import jax
import jax.numpy as jnp
from jax import lax
from jax.experimental import pallas as pl
from jax.experimental.pallas import tpu as pltpu
from jax.experimental.pallas import tpu_sc as plsc

D = 4
B = 4096
F = 1024
EPS = 1e-3

NC = 2
NS = 16
NW = NC * NS
ROWS = B // NW
CH = 16
NCH = ROWS // CH
LANES = 16
VPR = F // LANES
UNROLL = 8

_HI = -65536


def _tables_body(gg, gb, dg, db, mm, mv, st_ref):
    for d in range(D):
        s = (gg[0] * dg[d]) * lax.rsqrt(mv[d, :] + EPS)
        t = (gb[0] + db[d]) - s * mm[d, :]
        si = lax.bitcast_convert_type(s, jnp.int32)
        ti = lax.bitcast_convert_type(t, jnp.int32)
        sw = (si + 0x8000) & _HI
        tw = lax.shift_right_logical(ti + 0x8000, 16)
        st_ref[pl.ds(d * F, F)] = sw | tw


def _compute_tables(gg, gb, dg, db, mm, mv):
    return pl.pallas_call(
        _tables_body,
        in_specs=[
            pl.BlockSpec(memory_space=pltpu.SMEM),
            pl.BlockSpec(memory_space=pltpu.SMEM),
            pl.BlockSpec(memory_space=pltpu.SMEM),
            pl.BlockSpec(memory_space=pltpu.SMEM),
            pl.BlockSpec(),
            pl.BlockSpec(),
        ],
        out_shape=jax.ShapeDtypeStruct((D * F,), jnp.int32),
    )(gg, gb, dg, db, mm, mv)


def _sc_body(x_hbm, ids_hbm, st_hbm, out_hbm,
             ids_v, st_v, xbuf, obuf, offs_smem, in_sems, out_sems):
    wid = lax.axis_index("s") * NC + lax.axis_index("c")
    base = wid * ROWS

    for b in range(2):
        pltpu.async_copy(x_hbm.at[pl.ds(base + b * CH, CH)], xbuf.at[b],
                         in_sems.at[b])
    pltpu.sync_copy(ids_hbm.at[pl.ds(base, ROWS)], ids_v)
    pltpu.sync_copy(st_hbm, st_v)

    for k in range(NCH):
        dvec = ids_v[pl.ds(k * CH, CH)] * F
        for i in range(CH):
            offs_smem[k * CH + i] = dvec[i]

    def _compute_chunk(b, c):
        @plsc.parallel_loop(0, CH)
        def row_body(i, b=b, c=c):
            o = offs_smem[c * CH + i]

            @plsc.parallel_loop(0, VPR, unroll=UNROLL)
            def vec_body(j, i=i, o=o, b=b):
                off = j * LANES
                w = st_v[pl.ds(o + off, LANES)]
                sv = lax.bitcast_convert_type(w & _HI, jnp.float32)
                tv = lax.bitcast_convert_type(lax.shift_left(w, 16),
                                              jnp.float32)
                xv = xbuf[b, i, pl.ds(off, LANES)]
                obuf[b, i, pl.ds(off, LANES)] = xv * sv + tv

    def round_body(g, carry):
        for b in range(2):
            c = 2 * g + b
            r0 = base + c * CH
            pltpu.make_async_copy(x_hbm.at[pl.ds(r0, CH)], xbuf.at[b],
                                  in_sems.at[b]).wait()

            @pl.when(g > 0)
            def _(b=b, c=c):
                pltpu.make_async_copy(
                    obuf.at[b], out_hbm.at[pl.ds(base + (c - 2) * CH, CH)],
                    out_sems.at[b]).wait()

            _compute_chunk(b, c)
            pltpu.async_copy(obuf.at[b], out_hbm.at[pl.ds(r0, CH)],
                             out_sems.at[b])

            @pl.when(g < NCH // 2 - 1)
            def _(b=b, r0=r0):
                pltpu.async_copy(x_hbm.at[pl.ds(r0 + 2 * CH, CH)],
                                 xbuf.at[b], in_sems.at[b])

        return carry

    lax.fori_loop(0, NCH // 2, round_body, 0)
    for b in range(2):
        pltpu.make_async_copy(
            obuf.at[b], out_hbm.at[pl.ds(base + (NCH - 2 + b) * CH, CH)],
            out_sems.at[b]).wait()


@functools.partial(
    pl.kernel,
    out_type=jax.ShapeDtypeStruct((B, F), jnp.float32),
    mesh=plsc.VectorSubcoreMesh(core_axis_name="c", subcore_axis_name="s"),
    scratch_types=[
        pltpu.VMEM((ROWS,), jnp.int32),
        pltpu.VMEM((D * F,), jnp.int32),
        pltpu.VMEM((2, CH, F), jnp.float32),
        pltpu.VMEM((2, CH, F), jnp.float32),
        pltpu.SMEM((ROWS,), jnp.int32),
        pltpu.SemaphoreType.DMA((2,)),
        pltpu.SemaphoreType.DMA((2,)),
    ],
)
def _sc_apply(x_hbm, ids_hbm, st_hbm, out_hbm,
              ids_v, st_v, xbuf, obuf, offs_smem, in_sems, out_sems):
    _sc_body(x_hbm, ids_hbm, st_hbm, out_hbm,
             ids_v, st_v, xbuf, obuf, offs_smem, in_sems, out_sems)


def kernel(features, domain_types_idx, global_gamma, global_beta,
           domain_gammas, domain_betas, moving_means, moving_vars):
    st_tab = _compute_tables(global_gamma, global_beta,
                             domain_gammas, domain_betas,
                             moving_means, moving_vars)
    ids = domain_types_idx.reshape(-1)
    return _sc_apply(features, ids, st_tab)
